# baseline (speedup 1.0000x reference)
import jax
import jax.numpy as jnp
from jax import lax
from jax.experimental import pallas as pl
from jax.experimental.pallas import tpu as pltpu
from jax.experimental.pallas import tpu_sc as plsc

_CENTER_VARIANCE = 0.1
_SIZE_VARIANCE = 0.2
_IOU_THRESHOLD = 0.45
_SCORE_THRESHOLD = 0.01
_TOPK = 100

_B = 8
_N = 20000
_NP = 20480
_C = 80
_PAIRS = _B * _C
_KPAD = 112
_LANES = 16


_A = 2048


def _softmax_decode_body(cls_ref, box_ref, scores_ref, boxes_ref):
    i = pl.program_id(1)
    valid = (i * _A + lax.broadcasted_iota(jnp.int32, (_A, 1), 0)) < _N

    s = jnp.where(valid, cls_ref[0], 0.0)
    scores_ref[0] = s.T
    bxs = jnp.where(valid, box_ref[0], 0.0)
    boxes_ref[0] = bxs.T


def _softmax_decode(fg_scores, boxes):
    grid = (_B, _NP // _A)
    return pl.pallas_call(
        _softmax_decode_body,
        grid=grid,
        in_specs=[
            pl.BlockSpec((1, _A, _C), lambda b, i: (b, i, 0)),
            pl.BlockSpec((1, _A, 4), lambda b, i: (b, i, 0)),
        ],
        out_specs=[
            pl.BlockSpec((1, _C, _A), lambda b, i: (b, 0, i)),
            pl.BlockSpec((1, 4, _A), lambda b, i: (b, 0, i)),
        ],
        out_shape=[
            jax.ShapeDtypeStruct((_B, _C, _NP), jnp.float32),
            jax.ShapeDtypeStruct((_B, 4, _NP), jnp.float32),
        ],
    )(fg_scores, boxes)


_NCHUNK = _NP // _LANES
_HBITS1 = 11
_HBITS2 = 11
_HBITS3 = 10
_PAIRS_PER_TILE = _PAIRS // 32


def _sc_topk_body(scores_hbm, boxes_hbm, svals_hbm, sboxes_hbm,
                  scores_v, boxes_v, hist_v, cand_v, cand_i, eq_i,
                  outv_v, outi_v, outb_v):
    wid = lax.axis_index("s") * 2 + lax.axis_index("c")
    b = wid // 4
    cbase = (wid % 4) * _PAIRS_PER_TILE

    lanes = jax.lax.iota(jnp.int32, 16)
    ones_i = jnp.ones((16,), jnp.int32)
    zeros_i = jnp.zeros((16,), jnp.int32)
    neg_inf = jnp.full((16,), -jnp.inf, jnp.float32)
    lane0 = lanes == 0
    big_i = jnp.int32(2 ** 30)

    pltpu.sync_copy(boxes_hbm.at[b], boxes_v)

    def hist_clear(nbuck):
        def clr(j, _):
            hist_v[pl.ds(j * 16, 16)] = zeros_i
            return 0
        lax.fori_loop(0, nbuck // 16, clr, 0, unroll=8)

    def hist_pass(shift, nbuck, pshift, pval, use_prefix):
        def body(j, _):
            v = scores_v[pl.ds(j * 16, 16)]
            u = lax.bitcast_convert_type(v, jnp.int32)
            bk = jnp.bitwise_and(lax.shift_right_logical(u, shift),
                                 jnp.int32(nbuck - 1))
            if use_prefix:
                msk = lax.shift_right_logical(u, pshift) == pval
                plsc.addupdate_scatter(hist_v, [bk], ones_i, mask=msk)
            else:
                plsc.addupdate_scatter(hist_v, [bk], ones_i)
            return 0
        lax.fori_loop(0, _NCHUNK, body, 0, unroll=4)

    def hist_scan(nbuck, remaining):
        nv = nbuck // 16

        def body(jj, carry):
            best, rcb, cum = carry
            j = nv - 1 - jj
            h = hist_v[pl.ds(j * 16, 16)]
            cs = plsc.cumsum(lax.rev(h, (0,)))
            rc = lax.rev(cs, (0,)) + cum
            msk = rc >= remaining
            cand = jnp.where(msk, j * 16 + lanes, -1)
            best = jnp.maximum(best, jnp.max(cand))
            rcc = jnp.where(msk, rc, big_i)
            rcb = jnp.minimum(rcb, jnp.min(rcc))
            return best, rcb, cum + jnp.max(cs)

        best, rcb, _ = lax.fori_loop(
            0, nv, body, (jnp.int32(-1), big_i, jnp.int32(0)), unroll=4)
        hsel = jnp.max(plsc.load_gather(hist_v, [jnp.full((16,), best)]))
        return best, rcb, hsel

    def one_pair(t, _):
        p = b * _C + cbase + t
        pltpu.sync_copy(scores_hbm.at[p], scores_v)

        hist_clear(1 << _HBITS1)
        hist_pass(21, 1 << _HBITS1, 0, 0, False)
        b1, rc1, h1 = hist_scan(1 << _HBITS1, jnp.int32(_TOPK))
        rem2 = jnp.int32(_TOPK) - (rc1 - h1)

        hist_clear(1 << _HBITS2)
        hist_pass(10, 1 << _HBITS2, 21, b1, True)
        b2, rc2, h2 = hist_scan(1 << _HBITS2, rem2)
        rem3 = rem2 - (rc2 - h2)
        pref2 = jnp.bitwise_or(lax.shift_left(b1, 11), b2)

        hist_clear(1 << _HBITS3)
        hist_pass(0, 1 << _HBITS3, 10, pref2, True)
        b3, rc3, h3 = hist_scan(1 << _HBITS3, rem3)
        needed_eq = rem3 - (rc3 - h3)
        tbits = jnp.bitwise_or(lax.shift_left(pref2, 10), b3)

        def init_cand(j, _):
            cand_v[pl.ds(j * 16, 16)] = neg_inf
            return 0
        lax.fori_loop(0, 8, init_cand, 0, unroll=8)

        def collect(j, carry):
            cg, ce = carry
            v = scores_v[pl.ds(j * 16, 16)]
            u = lax.bitcast_convert_type(v, jnp.int32)
            gidx = j * 16 + lanes
            mgt = u > tbits
            posg = plsc.cumsum(jnp.where(mgt, 1, 0))
            dstg = cg + posg - 1
            plsc.store_scatter(cand_v, [dstg], v, mask=mgt)
            plsc.store_scatter(cand_i, [dstg], gidx, mask=mgt)
            cg = cg + jnp.max(plsc.all_reduce_population_count(mgt))
            meq = u == tbits
            pose = plsc.cumsum(jnp.where(meq, 1, 0))
            meq = meq & (ce + pose <= needed_eq)
            plsc.store_scatter(eq_i, [ce + pose - 1], gidx, mask=meq)
            ce = ce + jnp.max(plsc.all_reduce_population_count(meq))
            return cg, ce

        cg, ce = lax.fori_loop(0, _NCHUNK, collect,
                               (jnp.int32(0), jnp.int32(0)), unroll=4)

        tval = lax.bitcast_convert_type(
            jnp.full((16,), 0, jnp.int32) + tbits, jnp.float32)

        def put_eq(j, _):
            li = eq_i[pl.ds(j * 16, 16)]
            msk = (j * 16 + lanes) < needed_eq
            dst = cg + j * 16 + lanes
            plsc.store_scatter(cand_i, [dst], li, mask=msk)
            plsc.store_scatter(cand_v, [dst], tval, mask=msk)
            return 0
        lax.fori_loop(0, 7, put_eq, 0)

        def sel(i, _):
            mv = cand_v[pl.ds(0, 16)]
            for j in range(1, 7):
                mv = jnp.maximum(mv, cand_v[pl.ds(j * 16, 16)])
            mx = jnp.max(mv)
            bpos = big_i
            for j in range(7):
                vv = cand_v[pl.ds(j * 16, 16)]
                bpos = jnp.minimum(
                    bpos, jnp.min(jnp.where(vv == mx, j * 16 + lanes, big_i)))
            bidx = jnp.max(plsc.load_gather(cand_i, [jnp.full((16,), bpos)]))
            plsc.store_scatter(outv_v, [jnp.full((16,), i)],
                               jnp.full((16,), 0.0) + mx, mask=lane0)
            plsc.store_scatter(outi_v, [jnp.full((16,), i)],
                               zeros_i + bidx, mask=lane0)
            plsc.store_scatter(cand_v, [jnp.full((16,), bpos)], neg_inf,
                               mask=lane0)
            return 0
        lax.fori_loop(0, _TOPK, sel, 0)

        plsc.store_scatter(outv_v, [jnp.int32(_TOPK) + lanes],
                           jnp.zeros((16,), jnp.float32), mask=lanes < 12)
        plsc.store_scatter(outi_v, [jnp.int32(_TOPK) + lanes],
                           zeros_i, mask=lanes < 12)

        for j in range(7):
            idxv = outi_v[pl.ds(j * 16, 16)]
            for pln in range(4):
                g = plsc.load_gather(boxes_v, [jnp.full((16,), pln), idxv])
                outb_v[pln, pl.ds(j * 16, 16)] = g

        pltpu.sync_copy(outv_v, svals_hbm.at[p])
        pltpu.sync_copy(outb_v, sboxes_hbm.at[p])
        return 0

    lax.fori_loop(0, _PAIRS_PER_TILE, one_pair, 0)


def _sc_topk(scores_t, boxes_t):
    mesh = plsc.VectorSubcoreMesh(core_axis_name="c", subcore_axis_name="s",
                                  num_cores=2, num_subcores=16)
    f = pl.kernel(
        _sc_topk_body,
        out_type=[
            jax.ShapeDtypeStruct((_PAIRS, _KPAD), jnp.float32),
            jax.ShapeDtypeStruct((_PAIRS, 4, _KPAD), jnp.float32),
        ],
        mesh=mesh,
        compiler_params=pltpu.CompilerParams(needs_layout_passes=False),
        scratch_types=[
            pltpu.VMEM((_NP,), jnp.float32),
            pltpu.VMEM((4, _NP), jnp.float32),
            pltpu.VMEM((1 << _HBITS1,), jnp.int32),
            pltpu.VMEM((_KPAD + 16,), jnp.float32),
            pltpu.VMEM((_KPAD + 16,), jnp.int32),
            pltpu.VMEM((_KPAD + 16,), jnp.int32),
            pltpu.VMEM((_KPAD,), jnp.float32),
            pltpu.VMEM((_KPAD,), jnp.int32),
            pltpu.VMEM((4, _KPAD), jnp.float32),
        ],
    )
    return f(scores_t, boxes_t)


_G = 8


def _nms_body(svals_ref, sboxes_ref, dets_ref, iou_s):
    vals = svals_ref[...]
    x1 = sboxes_ref[:, 0, :]
    y1 = sboxes_ref[:, 1, :]
    x2 = sboxes_ref[:, 2, :]
    y2 = sboxes_ref[:, 3, :]

    for g in range(_G):
        ax1, ay1 = x1[g][:, None], y1[g][:, None]
        ax2, ay2 = x2[g][:, None], y2[g][:, None]
        bx1, by1 = x1[g][None, :], y1[g][None, :]
        bx2, by2 = x2[g][None, :], y2[g][None, :]
        iw = jnp.clip(jnp.minimum(ax2, bx2) - jnp.maximum(ax1, bx1), 0.0)
        ih = jnp.clip(jnp.minimum(ay2, by2) - jnp.maximum(ay1, by1), 0.0)
        inter = iw * ih
        aa = jnp.clip(ax2 - ax1, 0.0) * jnp.clip(ay2 - ay1, 0.0)
        ab = jnp.clip(bx2 - bx1, 0.0) * jnp.clip(by2 - by1, 0.0)
        iou = inter / (aa + ab - inter + 1e-9)
        iou_s[:, g, :] = iou

    lanei = lax.broadcasted_iota(jnp.int32, (_G, _KPAD), 1)

    def body(i, keep):
        row = iou_s[i]
        keep_i = jnp.sum(jnp.where(lanei == i, keep, 0.0), axis=1,
                         keepdims=True)
        sup = (row > _IOU_THRESHOLD) & (lanei > i) & (keep_i > 0.0)
        return jnp.where(sup, 0.0, keep)

    keep = lax.fori_loop(0, _TOPK, body, jnp.ones((_G, _KPAD), jnp.float32))
    m = keep * (vals > _SCORE_THRESHOLD)
    dets_ref[:, 0, :] = x1 * m
    dets_ref[:, 1, :] = y1 * m
    dets_ref[:, 2, :] = x2 * m
    dets_ref[:, 3, :] = y2 * m
    dets_ref[:, 4, :] = vals * m


def _nms(svals, sboxes):
    grid = (_PAIRS // _G,)
    return pl.pallas_call(
        _nms_body,
        grid=grid,
        in_specs=[
            pl.BlockSpec((_G, _KPAD), lambda i: (i, 0)),
            pl.BlockSpec((_G, 4, _KPAD), lambda i: (i, 0, 0)),
        ],
        out_specs=pl.BlockSpec((_G, 5, _KPAD), lambda i: (i, 0, 0)),
        out_shape=jax.ShapeDtypeStruct((_PAIRS, 5, _KPAD), jnp.float32),
        scratch_shapes=[pltpu.VMEM((_KPAD, _G, _KPAD), jnp.float32)],
    )(svals, sboxes)


def kernel(cls_logits, bbox_pred, priors):
    fg_scores = jax.nn.softmax(cls_logits, axis=2)[..., 1:]
    centers = (bbox_pred[..., :2] * _CENTER_VARIANCE * priors[..., 2:]
               + priors[..., :2])
    sizes = jnp.exp(bbox_pred[..., 2:] * _SIZE_VARIANCE) * priors[..., 2:]
    boxes = jnp.concatenate([centers - sizes / 2.0, centers + sizes / 2.0],
                            axis=-1)
    scores_t, boxes_t = _softmax_decode(fg_scores, boxes)
    svals, sboxes = _sc_topk(scores_t.reshape(_PAIRS, _NP), boxes_t)
    dets = _nms(svals, sboxes)
    return dets[:, :, :_TOPK].transpose(0, 2, 1).reshape(_B, _C, _TOPK, 5)
